# baseline (speedup 1.0000x reference)
import jax
import jax.numpy as jnp
from jax import lax
from jax.experimental import pallas as pl
from jax.experimental.pallas import tpu as pltpu
from jax.experimental.pallas import tpu_sc as plsc

B, L, D = 4096, 200, 64
NC, NS = 2, 16
NW = NC * NS
ROWS_PER_W = B // NW
ROW_F = L * D
RB = 4
NBLK = ROWS_PER_W // RB


def _body(idx_hbm, emb_hbm, out_hbm, idx_v, rows_v):
    wid = lax.axis_index("s") * NC + lax.axis_index("c")
    base = wid * ROWS_PER_W

    pltpu.sync_copy(idx_hbm.at[pl.ds(base * L, ROWS_PER_W * L)], idx_v)

    zero16 = jnp.zeros((16,), jnp.float32)

    def _block(blk, _):
        blk_row = blk * RB
        pltpu.sync_copy(emb_hbm.at[pl.ds((base + blk_row) * ROW_F, RB * ROW_F)],
                        rows_v)
        for rr in range(RB):
            r = blk_row + rr
            m = jnp.full((16,), L, jnp.int32)
            for i in range(13):
                o = 184 if i == 12 else i * 16
                v = idx_v[pl.ds(r * L + o, 16)]
                pos = lax.iota(jnp.int32, 16) + o
                m = jnp.minimum(m, jnp.where(v == 0, pos, L))
            cutoff = m[0]
            for j in range(1, 16):
                cutoff = jnp.minimum(cutoff, m[j])

            zbase = rr * ROW_F + cutoff * D

            def _zpos(p, _, zbase=zbase):
                for q in range(0, D, 16):
                    rows_v[pl.ds(zbase + p * D + q, 16)] = zero16
                return 0
            lax.fori_loop(0, L - cutoff, _zpos, 0)
        pltpu.sync_copy(rows_v,
                        out_hbm.at[pl.ds((base + blk_row) * ROW_F, RB * ROW_F)])
        return 0

    lax.fori_loop(0, NBLK, _block, 0)


@jax.jit
def _run(idx1, emb1):
    mesh = plsc.VectorSubcoreMesh(core_axis_name="c", subcore_axis_name="s",
                                  num_cores=NC, num_subcores=NS)
    return pl.kernel(
        _body,
        out_type=jax.ShapeDtypeStruct((B * ROW_F,), jnp.float32),
        mesh=mesh,
        scratch_types=[
            pltpu.VMEM((ROWS_PER_W * L,), jnp.int32),
            pltpu.VMEM((RB * ROW_F,), jnp.float32),
        ],
    )(idx1, emb1)


def kernel(indexes, embeds):
    idx1 = indexes.astype(jnp.int32).reshape(-1)
    emb1 = embeds.reshape(-1)
    out = _run(idx1, emb1)
    return out.reshape(B, L, D)


# speedup vs baseline: 1.0249x; 1.0249x over previous
import jax
import jax.numpy as jnp
from jax import lax
from jax.experimental import pallas as pl
from jax.experimental.pallas import tpu as pltpu
from jax.experimental.pallas import tpu_sc as plsc

B, L, D = 4096, 200, 64
NC, NS = 2, 16
NW = NC * NS
ROWS_PER_W = B // NW
ROW_F = L * D
RB = 1
NBLK = ROWS_PER_W // RB
NBUF = 4


def _body(idx_hbm, emb_hbm, out_hbm, idx_v, bufs, in_sems, out_sems):
    wid = lax.axis_index("s") * NC + lax.axis_index("c")
    base = wid * ROWS_PER_W

    pltpu.sync_copy(idx_hbm.at[pl.ds(base * L, ROWS_PER_W * L)], idx_v)

    def in_desc(blk, s):
        return pltpu.make_async_copy(
            emb_hbm.at[pl.ds((base + blk * RB) * ROW_F, RB * ROW_F)],
            bufs[s], in_sems[s])

    def out_desc(blk, s):
        return pltpu.make_async_copy(
            bufs[s], out_hbm.at[pl.ds((base + blk * RB) * ROW_F, RB * ROW_F)],
            out_sems[s])

    zero16 = jnp.zeros((16,), jnp.float32)

    def _compute(blk, s):
        for rr in range(RB):
            r = blk * RB + rr
            m = jnp.full((16,), L, jnp.int32)
            for i in range(13):
                o = 184 if i == 12 else i * 16
                v = idx_v[pl.ds(r * L + o, 16)]
                pos = lax.iota(jnp.int32, 16) + o
                m = jnp.minimum(m, jnp.where(v == 0, pos, L))
            cutoff = m[0]
            for j in range(1, 16):
                cutoff = jnp.minimum(cutoff, m[j])

            zbase = rr * ROW_F + cutoff * D

            def _zpos(p, _, zbase=zbase, s=s):
                for q in range(0, D, 16):
                    bufs[s][pl.ds(zbase + p * D + q, 16)] = zero16
                return 0
            lax.fori_loop(0, L - cutoff, _zpos, 0)

    in_desc(0, 0).start()

    def _group(g, _):
        for s in range(NBUF):
            blk = g * NBUF + s
            nxt = (s + 1) % NBUF

            @pl.when(blk + 1 < NBLK)
            def _pre(blk=blk, nxt=nxt):
                @pl.when(blk + 1 >= NBUF)
                def _drain(blk=blk, nxt=nxt):
                    out_desc(blk + 1 - NBUF, nxt).wait()
                in_desc(blk + 1, nxt).start()

            in_desc(blk, s).wait()
            _compute(blk, s)
            out_desc(blk, s).start()
        return 0

    lax.fori_loop(0, NBLK // NBUF, _group, 0)

    for s in range(NBUF):
        out_desc(NBLK - NBUF + s, s).wait()


@jax.jit
def _run(idx1, emb1):
    mesh = plsc.VectorSubcoreMesh(core_axis_name="c", subcore_axis_name="s",
                                  num_cores=NC, num_subcores=NS)

    def body(idx_hbm, emb_hbm, out_hbm, idx_v,
             b0, b1, b2, b3, i0, i1, i2, i3, o0, o1, o2, o3):
        _body(idx_hbm, emb_hbm, out_hbm, idx_v,
              (b0, b1, b2, b3), (i0, i1, i2, i3), (o0, o1, o2, o3))

    return pl.kernel(
        body,
        out_type=jax.ShapeDtypeStruct((B * ROW_F,), jnp.float32),
        mesh=mesh,
        scratch_types=(
            [pltpu.VMEM((ROWS_PER_W * L,), jnp.int32)]
            + [pltpu.VMEM((RB * ROW_F,), jnp.float32)] * NBUF
            + [pltpu.SemaphoreType.DMA] * (2 * NBUF)
        ),
    )(idx1, emb1)


def kernel(indexes, embeds):
    idx1 = indexes.astype(jnp.int32).reshape(-1)
    emb1 = embeds.reshape(-1)
    out = _run(idx1, emb1)
    return out.reshape(B, L, D)


# speedup vs baseline: 2.0919x; 2.0411x over previous
import jax
import jax.numpy as jnp
from jax import lax
from jax.experimental import pallas as pl
from jax.experimental.pallas import tpu as pltpu
from jax.experimental.pallas import tpu_sc as plsc

B, L, D = 4096, 200, 64
NC, NS = 2, 16
NW = NC * NS
ROWS_PER_W = B // NW
ROW_F = L * D
RB = 8
CB = 1280
NCB = ROW_F // CB
NRB = ROWS_PER_W // RB
NBLK = NRB * NCB
NBUF = 4
POS_PER_CB = CB // D


def _body(idx_hbm, emb_hbm, out_hbm, idx_v, bufs, cut_s, in_sems, out_sems):
    wid = lax.axis_index("s") * NC + lax.axis_index("c")
    base = wid * ROWS_PER_W

    pltpu.sync_copy(idx_hbm.at[pl.ds(base * L, ROWS_PER_W * L)], idx_v)

    def _cut(r, _):
        m = jnp.full((16,), L, jnp.int32)
        for i in range(13):
            o = 184 if i == 12 else i * 16
            v = idx_v[pl.ds(r * L + o, 16)]
            pos = lax.iota(jnp.int32, 16) + o
            m = jnp.minimum(m, jnp.where(v == 0, pos, L))
        cutoff = m[0]
        for j in range(1, 16):
            cutoff = jnp.minimum(cutoff, m[j])
        cut_s[r] = cutoff
        return 0
    lax.fori_loop(0, ROWS_PER_W, _cut, 0)

    def in_desc(blk, s):
        rb, cb = blk // NCB, blk % NCB
        return pltpu.make_async_copy(
            emb_hbm.at[pl.ds(base + rb * RB, RB), pl.ds(cb * CB, CB)],
            bufs[s], in_sems[s])

    def out_desc(blk, s):
        rb, cb = blk // NCB, blk % NCB
        return pltpu.make_async_copy(
            bufs[s], out_hbm.at[pl.ds(base + rb * RB, RB), pl.ds(cb * CB, CB)],
            out_sems[s])

    zero16 = jnp.zeros((16,), jnp.float32)

    def _compute(blk, s):
        rb, cb = blk // NCB, blk % NCB
        for r in range(RB):
            cutoff = cut_s[rb * RB + r]
            p0 = jnp.clip(cutoff - cb * POS_PER_CB, 0, POS_PER_CB)

            def _zpos(p, _, r=r, s=s):
                for q in range(0, D, 16):
                    bufs[s][r, pl.ds(p * D + q, 16)] = zero16
                return 0
            lax.fori_loop(p0, POS_PER_CB, _zpos, 0)

    in_desc(0, 0).start()

    def _group(g, _):
        for s in range(NBUF):
            blk = g * NBUF + s
            nxt = (s + 1) % NBUF

            @pl.when(blk + 1 < NBLK)
            def _pre(blk=blk, nxt=nxt):
                @pl.when(blk + 1 >= NBUF)
                def _drain(blk=blk, nxt=nxt):
                    out_desc(blk + 1 - NBUF, nxt).wait()
                in_desc(blk + 1, nxt).start()

            in_desc(blk, s).wait()
            _compute(blk, s)
            out_desc(blk, s).start()
        return 0

    lax.fori_loop(0, NBLK // NBUF, _group, 0)

    for s in range(NBUF):
        out_desc(NBLK - NBUF + s, s).wait()


@jax.jit
def _run(idx1, emb2):
    mesh = plsc.VectorSubcoreMesh(core_axis_name="c", subcore_axis_name="s",
                                  num_cores=NC, num_subcores=NS)

    def body(idx_hbm, emb_hbm, out_hbm, idx_v,
             b0, b1, b2, b3, cut_s, i0, i1, i2, i3, o0, o1, o2, o3):
        _body(idx_hbm, emb_hbm, out_hbm, idx_v,
              (b0, b1, b2, b3), cut_s, (i0, i1, i2, i3), (o0, o1, o2, o3))

    return pl.kernel(
        body,
        out_type=jax.ShapeDtypeStruct((B, ROW_F), jnp.float32),
        mesh=mesh,
        scratch_types=(
            [pltpu.VMEM((ROWS_PER_W * L,), jnp.int32)]
            + [pltpu.VMEM((RB, CB), jnp.float32)] * NBUF
            + [pltpu.SMEM((ROWS_PER_W,), jnp.int32)]
            + [pltpu.SemaphoreType.DMA] * (2 * NBUF)
        ),
    )(idx1, emb2)


def kernel(indexes, embeds):
    idx1 = indexes.astype(jnp.int32).reshape(-1)
    emb2 = embeds.reshape(B, ROW_F)
    out = _run(idx1, emb2)
    return out.reshape(B, L, D)


# speedup vs baseline: 2.0940x; 1.0010x over previous
import jax
import jax.numpy as jnp
from jax import lax
from jax.experimental import pallas as pl
from jax.experimental.pallas import tpu as pltpu
from jax.experimental.pallas import tpu_sc as plsc

B, L, D = 4096, 200, 64
NC, NS = 2, 16
NW = NC * NS
ROWS_PER_W = B // NW
ROW_F = L * D
RB = 8
CB = 1280
NCB = ROW_F // CB
NRB = ROWS_PER_W // RB
NBLK = NRB * NCB
NBUF = 4
POS_PER_CB = CB // D


def _body(idx_hbm, emb_hbm, out_hbm, idx_v, bufs, cut_s, in_sems, out_sems):
    wid = lax.axis_index("s") * NC + lax.axis_index("c")
    base = wid * ROWS_PER_W

    pltpu.sync_copy(idx_hbm.at[pl.ds(base * L, ROWS_PER_W * L)], idx_v)

    def _cut(r, _):
        m = jnp.full((16,), L, jnp.int32)
        for i in range(13):
            o = 184 if i == 12 else i * 16
            v = idx_v[pl.ds(r * L + o, 16)]
            pos = lax.iota(jnp.int32, 16) + o
            m = jnp.minimum(m, jnp.where(v == 0, pos, L))
        cutoff = m[0]
        for j in range(1, 16):
            cutoff = jnp.minimum(cutoff, m[j])
        cut_s[r] = cutoff
        return 0
    lax.fori_loop(0, ROWS_PER_W, _cut, 0)

    def in_desc(blk, s):
        rb, cb = blk // NCB, blk % NCB
        return pltpu.make_async_copy(
            emb_hbm.at[pl.ds(base + rb * RB, RB), pl.ds(cb * CB, CB)],
            bufs[s], in_sems[s])

    def out_desc(blk, s):
        rb, cb = blk // NCB, blk % NCB
        return pltpu.make_async_copy(
            bufs[s], out_hbm.at[pl.ds(base + rb * RB, RB), pl.ds(cb * CB, CB)],
            out_sems[s])

    zero16 = jnp.zeros((16,), jnp.float32)

    def _compute(blk, s):
        rb, cb = blk // NCB, blk % NCB
        for r in range(RB):
            cutoff = cut_s[rb * RB + r]
            p0 = jnp.clip(cutoff - cb * POS_PER_CB, 0, POS_PER_CB)

            def _zpos(p, _, r=r, s=s):
                for q in range(0, D, 16):
                    bufs[s][r, pl.ds(p * D + q, 16)] = zero16
                return 0
            lax.fori_loop(p0, POS_PER_CB, _zpos, 0)

    in_desc(0, 0).start()

    def _group(g, _):
        for s in range(NBUF):
            blk = g * NBUF + s
            nxt = (s + 1) % NBUF

            @pl.when(blk + 1 < NBLK)
            def _pre(blk=blk, nxt=nxt):
                @pl.when(blk + 1 >= NBUF)
                def _drain(blk=blk, nxt=nxt):
                    out_desc(blk + 1 - NBUF, nxt).wait()
                in_desc(blk + 1, nxt).start()

            in_desc(blk, s).wait()
            _compute(blk, s)
            out_desc(blk, s).start()
        return 0

    lax.fori_loop(0, NBLK // NBUF, _group, 0)

    for s in range(NBUF):
        out_desc(NBLK - NBUF + s, s).wait()


@jax.jit
def _run(idx1, emb2):
    mesh = plsc.VectorSubcoreMesh(core_axis_name="c", subcore_axis_name="s",
                                  num_cores=NC, num_subcores=NS)

    def body(idx_hbm, emb_hbm, out_hbm, idx_v,
             b0, b1, b2, b3, cut_s, i0, i1, i2, i3, o0, o1, o2, o3):
        _body(idx_hbm, emb_hbm, out_hbm, idx_v,
              (b0, b1, b2, b3), cut_s, (i0, i1, i2, i3), (o0, o1, o2, o3))

    return pl.kernel(
        body,
        out_type=jax.ShapeDtypeStruct((B, ROW_F), jnp.float32),
        mesh=mesh,
        scratch_types=(
            [pltpu.VMEM((ROWS_PER_W * L,), jnp.int32)]
            + [pltpu.VMEM((RB, CB), jnp.float32)] * NBUF
            + [pltpu.SMEM((ROWS_PER_W,), jnp.int32)]
            + [pltpu.SemaphoreType.DMA] * (2 * NBUF)
        ),
        compiler_params=pltpu.CompilerParams(use_tc_tiling_on_sc=True),
    )(idx1, emb2)


def kernel(indexes, embeds):
    idx1 = indexes.astype(jnp.int32).reshape(-1)
    emb2 = embeds.reshape(B, ROW_F)
    out = _run(idx1, emb2)
    return out.reshape(B, L, D)


# speedup vs baseline: 5.1625x; 2.4654x over previous
import jax
import jax.numpy as jnp
from jax import lax
from jax.experimental import pallas as pl
from jax.experimental.pallas import tpu as pltpu
from jax.experimental.pallas import tpu_sc as plsc

B, L, D = 4096, 200, 64
NC, NS = 2, 16
NW = NC * NS
BW = B // NW
F = L * D
FC = 160
NBLK = F // FC
NBUF = 4


def _body(idx_hbm, emb_hbm, out_hbm, idx_v, bufs, thr_s, lane_s, in_sems, out_sems):
    wid = lax.axis_index("s") * NC + lax.axis_index("c")
    b0 = wid * BW

    pltpu.sync_copy(idx_hbm.at[:, pl.ds(b0, BW)], idx_v)

    def _scan_l(l, ms):
        lvec = jnp.full((16,), 0, jnp.int32) + l
        return tuple(
            jnp.minimum(ms[g],
                        jnp.where(idx_v[l, pl.ds(g * 16, 16)] == 0, lvec, L))
            for g in range(8))
    ms = lax.fori_loop(0, L, _scan_l,
                       tuple(jnp.full((16,), L, jnp.int32) for _ in range(8)))

    np_ = jnp.int32(0)
    for g in range(8):
        for j in range(16):
            c = ms[g][j]
            bl = g * 16 + j
            thr_s[bl] = c * D

            @pl.when(c < L)
            def _add(np_=np_, bl=bl):
                lane_s[np_] = bl
            np_ = jnp.where(c < L, np_ + 1, np_)

    def in_desc(blk, s):
        return pltpu.make_async_copy(
            emb_hbm.at[pl.ds(blk * FC, FC), pl.ds(b0, BW)],
            bufs[s], in_sems[s])

    def out_desc(blk, s):
        return pltpu.make_async_copy(
            bufs[s], out_hbm.at[pl.ds(blk * FC, FC), pl.ds(b0, BW)],
            out_sems[s])

    zero16 = jnp.zeros((16,), jnp.float32)
    iota16 = lax.iota(jnp.int32, 16)

    def _compute(blk, s):
        f0 = blk * FC

        def _lane(i, _):
            bl = lane_s[i]
            lo = jnp.maximum(thr_s[bl] - f0, 0)

            @pl.when(lo < FC)
            def _zero(bl=bl, lo=lo):
                bvec = jnp.full((16,), 0, jnp.int32) + bl

                def _chunk(k, _):
                    fvals = lo + k * 16 + iota16
                    plsc.store_scatter(bufs[s], [fvals, bvec], zero16,
                                       mask=fvals < FC)
                    return 0
                lax.fori_loop(0, (FC - lo + 15) >> 4, _chunk, 0)
            return 0
        lax.fori_loop(0, np_, _lane, 0)

    in_desc(0, 0).start()

    def _group(g, _):
        for s in range(NBUF):
            blk = g * NBUF + s
            nxt = (s + 1) % NBUF

            @pl.when(blk + 1 < NBLK)
            def _pre(blk=blk, nxt=nxt):
                @pl.when(blk + 1 >= NBUF)
                def _drain(blk=blk, nxt=nxt):
                    out_desc(blk + 1 - NBUF, nxt).wait()
                in_desc(blk + 1, nxt).start()

            in_desc(blk, s).wait()
            _compute(blk, s)
            out_desc(blk, s).start()
        return 0

    lax.fori_loop(0, NBLK // NBUF, _group, 0)

    for s in range(NBUF):
        out_desc(NBLK - NBUF + s, s).wait()


@jax.jit
def _run(idx_t, emb_t):
    mesh = plsc.VectorSubcoreMesh(core_axis_name="c", subcore_axis_name="s",
                                  num_cores=NC, num_subcores=NS)

    def body(idx_hbm, emb_hbm, out_hbm, idx_v,
             b0_, b1_, b2_, b3_, thr_s, lane_s, i0, i1, i2, i3, o0, o1, o2, o3):
        _body(idx_hbm, emb_hbm, out_hbm, idx_v,
              (b0_, b1_, b2_, b3_), thr_s, lane_s,
              (i0, i1, i2, i3), (o0, o1, o2, o3))

    return pl.kernel(
        body,
        out_type=jax.ShapeDtypeStruct((F, B), jnp.float32),
        mesh=mesh,
        scratch_types=(
            [pltpu.VMEM((L, BW), jnp.int32)]
            + [pltpu.VMEM((FC, BW), jnp.float32)] * NBUF
            + [pltpu.SMEM((BW,), jnp.int32), pltpu.SMEM((BW,), jnp.int32)]
            + [pltpu.SemaphoreType.DMA] * (2 * NBUF)
        ),
        compiler_params=pltpu.CompilerParams(needs_layout_passes=False),
    )(idx_t, emb_t)


def kernel(indexes, embeds):
    idx_t = indexes.astype(jnp.int32).transpose(1, 0)
    emb_t = embeds.transpose(1, 2, 0).reshape(F, B)
    out_t = _run(idx_t, emb_t)
    return out_t.reshape(L, D, B).transpose(2, 0, 1)


# speedup vs baseline: 5.4453x; 1.0548x over previous
import jax
import jax.numpy as jnp
from jax import lax
from jax.experimental import pallas as pl
from jax.experimental.pallas import tpu as pltpu
from jax.experimental.pallas import tpu_sc as plsc

B, L, D = 4096, 200, 64
NC, NS = 2, 16
NW = NC * NS
BW = B // NW
F = L * D
FC = 200
NBLK = F // FC
NBUF = 4


def _body(idx_hbm, emb_hbm, out_hbm, idx_v, bufs, thr_s, lane_s, in_sems, out_sems):
    wid = lax.axis_index("s") * NC + lax.axis_index("c")
    b0 = wid * BW

    def in_desc(blk, s):
        return pltpu.make_async_copy(
            emb_hbm.at[pl.ds(blk * FC, FC), pl.ds(b0, BW)],
            bufs[s], in_sems[s])

    def out_desc(blk, s):
        return pltpu.make_async_copy(
            bufs[s], out_hbm.at[pl.ds(blk * FC, FC), pl.ds(b0, BW)],
            out_sems[s])

    in_desc(0, 0).start()
    in_desc(1, 1).start()

    pltpu.sync_copy(idx_hbm.at[:, pl.ds(b0, BW)], idx_v)

    def _scan_l(l, ms):
        lvec = jnp.full((16,), 0, jnp.int32) + l
        return tuple(
            jnp.minimum(ms[g],
                        jnp.where(idx_v[l, pl.ds(g * 16, 16)] == 0, lvec, L))
            for g in range(8))
    ms = lax.fori_loop(0, L, _scan_l,
                       tuple(jnp.full((16,), L, jnp.int32) for _ in range(8)))

    np_ = jnp.int32(0)
    for g in range(8):
        for j in range(16):
            c = ms[g][j]
            bl = g * 16 + j
            thr_s[bl] = c * D

            @pl.when(c < L)
            def _add(np_=np_, bl=bl):
                lane_s[np_] = bl
            np_ = jnp.where(c < L, np_ + 1, np_)

    zero16 = jnp.zeros((16,), jnp.float32)
    iota16 = lax.iota(jnp.int32, 16)

    def _compute(blk, s):
        f0 = blk * FC

        def _lane(i, _):
            bl = lane_s[i]
            lo = jnp.maximum(thr_s[bl] - f0, 0)

            @pl.when(lo < FC)
            def _zero(bl=bl, lo=lo):
                bvec = jnp.full((16,), 0, jnp.int32) + bl

                def _chunk(k, _):
                    fvals = lo + k * 16 + iota16
                    plsc.store_scatter(bufs[s], [fvals, bvec], zero16,
                                       mask=fvals < FC)
                    return 0
                lax.fori_loop(0, (FC - lo + 15) >> 4, _chunk, 0)
            return 0
        lax.fori_loop(0, np_, _lane, 0)

    def _group(g, _):
        for s in range(NBUF):
            blk = g * NBUF + s
            nxt = (s + 2) % NBUF

            @pl.when(blk + 2 < NBLK)
            def _pre(blk=blk, nxt=nxt):
                @pl.when(blk + 2 >= NBUF)
                def _drain(blk=blk, nxt=nxt):
                    out_desc(blk + 2 - NBUF, nxt).wait()
                in_desc(blk + 2, nxt).start()

            in_desc(blk, s).wait()
            _compute(blk, s)
            out_desc(blk, s).start()
        return 0

    lax.fori_loop(0, NBLK // NBUF, _group, 0)

    out_desc(NBLK - 2, (NBLK - 2) % NBUF).wait()
    out_desc(NBLK - 1, (NBLK - 1) % NBUF).wait()


@jax.jit
def _run(idx_t, emb_t):
    mesh = plsc.VectorSubcoreMesh(core_axis_name="c", subcore_axis_name="s",
                                  num_cores=NC, num_subcores=NS)

    def body(idx_hbm, emb_hbm, out_hbm, idx_v,
             b0_, b1_, b2_, b3_, thr_s, lane_s, i0, i1, i2, i3, o0, o1, o2, o3):
        _body(idx_hbm, emb_hbm, out_hbm, idx_v,
              (b0_, b1_, b2_, b3_), thr_s, lane_s,
              (i0, i1, i2, i3), (o0, o1, o2, o3))

    return pl.kernel(
        body,
        out_type=jax.ShapeDtypeStruct((F, B), jnp.float32),
        mesh=mesh,
        scratch_types=(
            [pltpu.VMEM((L, BW), jnp.int32)]
            + [pltpu.VMEM((FC, BW), jnp.float32)] * NBUF
            + [pltpu.SMEM((BW,), jnp.int32), pltpu.SMEM((BW,), jnp.int32)]
            + [pltpu.SemaphoreType.DMA] * (2 * NBUF)
        ),
        compiler_params=pltpu.CompilerParams(needs_layout_passes=False),
    )(idx_t, emb_t)


def kernel(indexes, embeds):
    idx_t = indexes.astype(jnp.int32).transpose(1, 0)
    emb_t = embeds.transpose(1, 2, 0).reshape(F, B)
    out_t = _run(idx_t, emb_t)
    return out_t.reshape(L, D, B).transpose(2, 0, 1)
